# Initial kernel scaffold; baseline (speedup 1.0000x reference)
import jax
import jax.numpy as jnp
from jax.experimental import pallas as pl


def kernel(edge_index, graph_embedding, weight):
    raise NotImplementedError("write your pallas kernel here")


# speedup vs baseline: 8.2141x; 8.2141x over previous
import functools

import jax
import jax.numpy as jnp
from jax import lax
from jax.experimental import pallas as pl
from jax.experimental.pallas import tpu as pltpu
from jax.experimental.pallas import tpu_sc as plsc

N_NODES = 10000
N_EDGES = 320000
D = 128
NC = 2
NS = 16
NW = NC * NS
EPT = N_EDGES // NW
CH = 100
K = EPT // CH
NP = 10240
RPT = NP // NS
ZR = 40


def _elu_body(g_ref, w_ref, out_ref):
    x = g_ref[...] * w_ref[...]
    out_ref[...] = jnp.where(x > 0, x, jnp.exp(jnp.minimum(x, 0.0)) - 1.0)


def _add_body(p_ref, out_ref):
    out_ref[...] = p_ref[0, :N_NODES] + p_ref[1, :N_NODES]


def _sc_body(src_hbm, dst_hbm, emb_hbm, out_hbm, src_v, dst_v, rows_v, zbuf,
             acc, sem):
    cid = lax.axis_index("c")
    sid = lax.axis_index("s")
    wid = cid * NS + sid

    def zstore(t, carry):
        i = t // (D // 16)
        j = t % (D // 16)
        zbuf[i, pl.ds(j * 16, 16)] = jnp.zeros((16,), jnp.float32)
        return carry

    lax.fori_loop(0, ZR * (D // 16), zstore, 0)

    row0 = sid * RPT
    for r in range(RPT // ZR):
        pltpu.sync_copy(zbuf, acc.at[pl.ds(row0 + r * ZR, ZR)])
    plsc.subcore_barrier()

    pltpu.sync_copy(src_hbm.at[wid], src_v)
    pltpu.sync_copy(dst_hbm.at[wid], dst_v)

    def chunk(j, carry):
        pltpu.async_copy(emb_hbm.at[src_v.at[j]], rows_v, sem).wait()
        pltpu.sync_copy(rows_v, acc.at[dst_v.at[j]], add=True)
        return carry

    lax.fori_loop(0, K, chunk, 0)

    plsc.subcore_barrier()
    pltpu.sync_copy(acc.at[pl.ds(row0, RPT)],
                    out_hbm.at[cid, pl.ds(row0, RPT)])


_sc_scatter = functools.partial(
    pl.kernel,
    out_type=jax.ShapeDtypeStruct((NC, NP, D), jnp.float32),
    mesh=plsc.VectorSubcoreMesh(core_axis_name="c", subcore_axis_name="s"),
    scratch_types=[
        pltpu.VMEM((K, CH), jnp.int32),
        pltpu.VMEM((K, CH), jnp.int32),
        pltpu.VMEM((CH, D), jnp.float32),
        pltpu.VMEM((ZR, D), jnp.float32),
        pltpu.VMEM_SHARED((NP, D), jnp.float32),
        pltpu.SemaphoreType.DMA,
    ],
)(_sc_body)


def kernel(edge_index, graph_embedding, weight):
    ei = edge_index.astype(jnp.int32)
    src = ei[0].reshape(NW, K, CH)
    dst = ei[1].reshape(NW, K, CH)

    emb = pl.pallas_call(
        _elu_body,
        out_shape=jax.ShapeDtypeStruct((N_NODES, D), jnp.float32),
    )(graph_embedding, weight)

    partials = _sc_scatter(src, dst, emb)

    out = pl.pallas_call(
        _add_body,
        out_shape=jax.ShapeDtypeStruct((N_NODES, D), jnp.float32),
    )(partials)
    return out
